# baseline (speedup 1.0000x reference)
import functools

import jax
import jax.numpy as jnp
from jax import lax
from jax.experimental import pallas as pl
from jax.experimental.pallas import tpu as pltpu
from jax.experimental.pallas import tpu_sc as plsc

N, E, NF, HID, NC, NL, NG = 10000, 320000, 128, 128, 10, 3, 64
BN_EPS = 1e-5

BLK = 1000
NBLK = N // BLK

NCORE, NSUB = 2, 16
NWORK = NCORE * NSUB
CH = 128
E_PAD = 327680
E_HALF = E_PAD // NCORE
PAD_ROWS = 16
OWN = 640
ECH = 2048
NECH = E_HALF // ECH
PLCAP = 896
NSLOT = 16 * PLCAP
CAPT = NSLOT + 16
NCHT = NSLOT // CH
LISTC = NSUB * CAPT
SMASK = (1 << 14) - 1


def _agg_body(h_hbm, src_hbm, dst_hbm, agg0, agg1,
              esrc_v, edst_v, pos_b, val_b, pk_v, idxb, patt, rows, acc, list_sh,
              gsem, psem, s0, s1, s2, s3):
    cid = lax.axis_index("c")
    sid = lax.axis_index("s")
    row0 = sid * OWN
    coff = cid * E_HALF
    tbase = sid * CAPT
    sems = [s0, s1, s2, s3]

    zero16 = jnp.zeros((16,), jnp.float32)
    lanes = lax.broadcasted_iota(jnp.int32, (16,), 0)

    def zrow(r, carry):
        for kk in range(HID // 16):
            acc[r, pl.ds(kk * 16, 16)] = zero16
        return carry

    lax.fori_loop(0, OWN + 1, zrow, 0)

    def prow(t, carry):
        v = (((t * 16 + lanes) * 19) & 8191) | (OWN << 14)
        patt[pl.ds(t * 16, 16)] = v
        return carry

    lax.fori_loop(0, 32, prow, 0)
    pre = []
    for t in range(NSLOT // 512):
        pre.append(pltpu.async_copy(
            patt, list_sh.at[pl.ds(tbase + t * 512, 512)], psem))
    for cp in pre:
        cp.wait()

    def scan_iter(it, cnts):
        off = coff + it * ECH
        pltpu.sync_copy(src_hbm.at[pl.ds(off, ECH)], esrc_v)
        pltpu.sync_copy(dst_hbm.at[pl.ds(off, ECH)], edst_v)
        for sc in range(4):
            for r in range(4):
                @pl.when(it > 0)
                def _():
                    pltpu.make_async_copy(
                        val_b.at[4 * sc + r],
                        list_sh.at[pos_b.at[4 * sc + r]],
                        sems[sc]).wait()

            def step(i2, cnts):
                base_e = sc * 512 + i2 * 16
                s16 = esrc_v[pl.ds(base_e, 16)]
                local = edst_v[pl.ds(base_e, 16)] - row0
                m = (local >= 0) & (local < OWN)
                pos16 = jnp.where(m, tbase + lanes * PLCAP + cnts,
                                  tbase + NSLOT + lanes)
                packed = s16 | (jnp.where(m, local, OWN) << 14)
                pos_b[4 * sc + i2 // 8, pl.ds((i2 % 8) * 16, 16)] = pos16
                val_b[4 * sc + i2 // 8, pl.ds((i2 % 8) * 16, 16)] = packed
                return cnts + jnp.where(m, 1, 0)

            cnts = lax.fori_loop(0, 32, step, cnts)
            for r in range(4):
                pltpu.async_copy(val_b.at[4 * sc + r],
                                 list_sh.at[pos_b.at[4 * sc + r]],
                                 sems[sc])
        return cnts

    lax.fori_loop(0, NECH, scan_iter, jnp.zeros((16,), jnp.int32))
    for sc in range(4):
        for r in range(4):
            pltpu.make_async_copy(val_b.at[4 * sc + r],
                                  list_sh.at[pos_b.at[4 * sc + r]],
                                  sems[sc]).wait()

    def accum_chunk(k, carry):
        base = tbase + k * CH
        pltpu.sync_copy(list_sh.at[pl.ds(base, CH)], pk_v)
        for o in range(CH // 16):
            idxb[pl.ds(o * 16, 16)] = pk_v[pl.ds(o * 16, 16)] & SMASK
        pltpu.async_copy(h_hbm.at[idxb], rows, gsem).wait()

        for g in range(CH // 16):
            d16 = jnp.minimum(pk_v[pl.ds(g * 16, 16)] >> 14, OWN)
            for jj in range(16):
                d = d16[jj]
                j = g * 16 + jj
                for kk in range(HID // 16):
                    sl = pl.ds(kk * 16, 16)
                    plsc.addupdate(acc.at[d, sl], rows[j, sl])
        return carry

    lax.fori_loop(0, NCHT, accum_chunk, 0)

    @pl.when(cid == 0)
    def _():
        @pl.when(sid < NSUB - 1)
        def _():
            pltpu.sync_copy(acc.at[pl.ds(0, OWN)],
                            agg0.at[pl.ds(row0, OWN)])

        @pl.when(sid == NSUB - 1)
        def _():
            pltpu.sync_copy(acc.at[pl.ds(0, N - (NSUB - 1) * OWN)],
                            agg0.at[pl.ds((NSUB - 1) * OWN,
                                          N - (NSUB - 1) * OWN)])

    @pl.when(cid == 1)
    def _():
        @pl.when(sid < NSUB - 1)
        def _():
            pltpu.sync_copy(acc.at[pl.ds(0, OWN)],
                            agg1.at[pl.ds(row0, OWN)])

        @pl.when(sid == NSUB - 1)
        def _():
            pltpu.sync_copy(acc.at[pl.ds(0, N - (NSUB - 1) * OWN)],
                            agg1.at[pl.ds((NSUB - 1) * OWN,
                                          N - (NSUB - 1) * OWN)])


@functools.cache
def _make_agg():
    return pl.kernel(
        _agg_body,
        out_type=[
            jax.ShapeDtypeStruct((N, HID), jnp.float32),
            jax.ShapeDtypeStruct((N, HID), jnp.float32),
        ],
        mesh=plsc.VectorSubcoreMesh(core_axis_name="c", subcore_axis_name="s"),
        scratch_types=[
            pltpu.VMEM((ECH,), jnp.int32),
            pltpu.VMEM((ECH,), jnp.int32),
            pltpu.VMEM((16, CH), jnp.int32),
            pltpu.VMEM((16, CH), jnp.int32),
            pltpu.VMEM((CH,), jnp.int32),
            pltpu.VMEM((CH,), jnp.int32),
            pltpu.VMEM((512,), jnp.int32),
            pltpu.VMEM((CH, HID), jnp.float32),
            pltpu.VMEM((OWN + 1, HID), jnp.float32),
            pltpu.VMEM_SHARED((LISTC,), jnp.int32),
            pltpu.SemaphoreType.DMA,
            pltpu.SemaphoreType.DMA,
            pltpu.SemaphoreType.DMA,
            pltpu.SemaphoreType.DMA,
            pltpu.SemaphoreType.DMA,
            pltpu.SemaphoreType.DMA,
        ],
    )


def _dot(a, b):
    return jnp.dot(a, b, preferred_element_type=jnp.float32,
                   precision=lax.Precision.HIGHEST)


def _mlp_block(z, w1, b1, w2, b2, g, be):
    z = jnp.maximum(_dot(z, w1) + b1, 0.0)
    z = jnp.maximum(_dot(z, w2) + b2, 0.0)
    return z * g + be


def _layer_body(h_ref, a0_ref, a1_ref, w1_ref, b1_ref, w2_ref, b2_ref,
                g_ref, be_ref, o_ref):
    z = h_ref[...] + a0_ref[...] + a1_ref[...]
    o_ref[...] = _mlp_block(z, w1_ref[...], b1_ref[...], w2_ref[...],
                            b2_ref[...], g_ref[...], be_ref[...])


def _row_spec():
    return pl.BlockSpec((BLK, HID), lambda i: (i, 0))


def _full_spec(shape):
    return pl.BlockSpec(shape, lambda i: (0,) * len(shape))


_layer_call = pl.pallas_call(
    _layer_body,
    grid=(NBLK,),
    in_specs=[_row_spec(), _row_spec(), _row_spec(),
              _full_spec((HID, HID)), _full_spec((1, HID)),
              _full_spec((HID, HID)), _full_spec((1, HID)),
              _full_spec((1, HID)), _full_spec((1, HID))],
    out_specs=_row_spec(),
    out_shape=jax.ShapeDtypeStruct((N, HID), jnp.float32),
)


def _final_body(h_ref, a0_ref, a1_ref, w1_ref, b1_ref, w2_ref, b2_ref,
                g_ref, be_ref, batch_ref, l1w_ref, l1b_ref, l2w_ref, l2b_ref,
                emb_ref, logp_ref, out_ref, pool_acc, cnt_acc):
    i = pl.program_id(0)

    @pl.when(i == 0)
    def _():
        pool_acc[...] = jnp.zeros_like(pool_acc)
        cnt_acc[...] = jnp.zeros_like(cnt_acc)

    z = h_ref[...] + a0_ref[...] + a1_ref[...]
    h3 = _mlp_block(z, w1_ref[...], b1_ref[...], w2_ref[...], b2_ref[...],
                    g_ref[...], be_ref[...])
    emb_ref[...] = h3

    bt = batch_ref[...].reshape(1, BLK)
    seg = lax.broadcasted_iota(jnp.int32, (NG, BLK), 0)
    mask = (bt == seg).astype(jnp.float32)
    pool_acc[...] += lax.dot_general(
        mask, h3, (((1,), (0,)), ((), ())),
        preferred_element_type=jnp.float32, precision=lax.Precision.HIGHEST)
    cnt_acc[...] += jnp.broadcast_to(
        jnp.sum(mask, axis=1, keepdims=True), (NG, HID))

    @pl.when(i == NBLK - 1)
    def _():
        pooled = pool_acc[...] / jnp.maximum(cnt_acc[...], 1.0)
        z1 = _dot(pooled, l1w_ref[...]) + l1b_ref[...]
        z2 = _dot(z1, l2w_ref[...]) + l2b_ref[...]
        outv = jnp.clip(z2, -10.0, 10.0)
        out_ref[...] = outv
        lane = lax.broadcasted_iota(jnp.int32, (NG, HID), 1)
        masked = jnp.where(lane < NC, outv, -1e30)
        m = jnp.max(masked, axis=1, keepdims=True)
        lse = jnp.log(jnp.sum(jnp.exp(masked - m), axis=1, keepdims=True)) + m
        logp_ref[...] = outv - lse


_final_call = pl.pallas_call(
    _final_body,
    grid=(NBLK,),
    in_specs=[_row_spec(), _row_spec(), _row_spec(),
              _full_spec((HID, HID)), _full_spec((1, HID)),
              _full_spec((HID, HID)), _full_spec((1, HID)),
              _full_spec((1, HID)), _full_spec((1, HID)),
              pl.BlockSpec((1, 1, BLK), lambda i: (i, 0, 0)),
              _full_spec((HID, HID)), _full_spec((1, HID)),
              _full_spec((HID, HID)), _full_spec((1, HID))],
    out_specs=[_row_spec(), _full_spec((NG, HID)), _full_spec((NG, HID))],
    out_shape=[jax.ShapeDtypeStruct((N, HID), jnp.float32),
               jax.ShapeDtypeStruct((NG, HID), jnp.float32),
               jax.ShapeDtypeStruct((NG, HID), jnp.float32)],
    scratch_shapes=[pltpu.VMEM((NG, HID), jnp.float32),
                    pltpu.VMEM((NG, HID), jnp.float32)],
)

_BN_INV = 1.0 / (1.0 + BN_EPS) ** 0.5


def kernel(x, edge_index, batch, params):
    src = edge_index[0]
    dst = edge_index[1]
    npad = E_PAD - E
    pad_idx = jnp.arange(npad, dtype=jnp.int32)
    src_p = jnp.concatenate([src, pad_idx % N])
    dst_p = jnp.concatenate([dst, N + (pad_idx % PAD_ROWS)])
    batch3 = batch.reshape(NBLK, 1, BLK)

    def layer_args(p):
        return (p['W1'], p['b1'].reshape(1, HID), p['W2'],
                p['b2'].reshape(1, HID),
                (p['gamma'] * _BN_INV).reshape(1, HID),
                p['beta'].reshape(1, HID))

    h = x
    for i in range(NL - 1):
        a0, a1 = _make_agg()(h, src_p, dst_p)
        h = _layer_call(h, a0, a1, *layer_args(params['c%d' % i]))

    a0, a1 = _make_agg()(h, src_p, dst_p)
    l2w = jnp.zeros((HID, HID), jnp.float32).at[:, :NC].set(params['lin2_W'])
    l2b = jnp.zeros((1, HID), jnp.float32).at[0, :NC].set(params['lin2_b'])
    embeds, logp128, out128 = _final_call(
        h, a0, a1, *layer_args(params['c%d' % (NL - 1)]),
        batch3, params['lin1_W'], params['lin1_b'].reshape(1, HID), l2w, l2b)
    return (logp128[:, :NC], embeds, out128[:, :NC])


# speedup vs baseline: 1.3932x; 1.3932x over previous
import functools

import jax
import jax.numpy as jnp
from jax import lax
from jax.experimental import pallas as pl
from jax.experimental.pallas import tpu as pltpu
from jax.experimental.pallas import tpu_sc as plsc

N, E, NF, HID, NC, NL, NG = 10000, 320000, 128, 128, 10, 3, 64
BN_EPS = 1e-5

BLK = 1000
NBLK = N // BLK

NCORE, NSUB = 2, 16
NWORK = NCORE * NSUB
CH = 128
E_PAD = 327680
E_HALF = E_PAD // NCORE
PAD_ROWS = 16
OWN = 640
ECH = 2048
NECH = E_HALF // ECH
PLCAP = 896
NSLOT = 16 * PLCAP
CAPT = NSLOT + 16
NCHT = NSLOT // CH
HC = 64
NHC = NSLOT // HC
LISTC = NSUB * CAPT
SMASK = (1 << 14) - 1


def _agg_body(h_hbm, src_hbm, dst_hbm, agg0, agg1,
              esrc_v, edst_v, pos_b, val_b, pk2, idxb2, patt, rows2, acc, list_sh,
              gs0, gs1, psem, s0, s1, s2, s3):
    cid = lax.axis_index("c")
    sid = lax.axis_index("s")
    row0 = sid * OWN
    coff = cid * E_HALF
    tbase = sid * CAPT
    sems = [s0, s1, s2, s3]
    gs = [gs0, gs1]

    zero16 = jnp.zeros((16,), jnp.float32)
    lanes = lax.broadcasted_iota(jnp.int32, (16,), 0)

    def zrow(r, carry):
        for kk in range(HID // 16):
            acc[r, pl.ds(kk * 16, 16)] = zero16
        return carry

    lax.fori_loop(0, OWN + 1, zrow, 0)

    def prow(t, carry):
        v = (((t * 16 + lanes) * 19) & 8191) | (OWN << 14)
        patt[pl.ds(t * 16, 16)] = v
        return carry

    lax.fori_loop(0, 32, prow, 0)
    pre = []
    for t in range(NSLOT // 512):
        pre.append(pltpu.async_copy(
            patt, list_sh.at[pl.ds(tbase + t * 512, 512)], psem))
    for cp in pre:
        cp.wait()

    def scan_iter(it, cnts):
        off = coff + it * ECH
        pltpu.sync_copy(src_hbm.at[pl.ds(off, ECH)], esrc_v)
        pltpu.sync_copy(dst_hbm.at[pl.ds(off, ECH)], edst_v)
        for sc in range(4):
            for r in range(4):
                @pl.when(it > 0)
                def _():
                    pltpu.make_async_copy(
                        val_b.at[4 * sc + r],
                        list_sh.at[pos_b.at[4 * sc + r]],
                        sems[sc]).wait()

            def step(i2, cnts):
                base_e = sc * 512 + i2 * 16
                s16 = esrc_v[pl.ds(base_e, 16)]
                local = edst_v[pl.ds(base_e, 16)] - row0
                m = (local >= 0) & (local < OWN)
                pos16 = jnp.where(m, tbase + lanes * PLCAP + cnts,
                                  tbase + NSLOT + lanes)
                packed = s16 | (jnp.where(m, local, OWN) << 14)
                pos_b[4 * sc + i2 // 8, pl.ds((i2 % 8) * 16, 16)] = pos16
                val_b[4 * sc + i2 // 8, pl.ds((i2 % 8) * 16, 16)] = packed
                return cnts + jnp.where(m, 1, 0)

            cnts = lax.fori_loop(0, 32, step, cnts)
            for r in range(4):
                pltpu.async_copy(val_b.at[4 * sc + r],
                                 list_sh.at[pos_b.at[4 * sc + r]],
                                 sems[sc])
        return cnts

    lax.fori_loop(0, NECH, scan_iter, jnp.zeros((16,), jnp.int32))
    for sc in range(4):
        for r in range(4):
            pltpu.make_async_copy(val_b.at[4 * sc + r],
                                  list_sh.at[pos_b.at[4 * sc + r]],
                                  sems[sc]).wait()

    def fire(hc, b):
        base = tbase + hc * HC
        pltpu.sync_copy(list_sh.at[pl.ds(base, HC)], pk2.at[b])
        for o in range(HC // 16):
            idxb2[b, pl.ds(o * 16, 16)] = pk2[b, pl.ds(o * 16, 16)] & SMASK
        pltpu.async_copy(h_hbm.at[idxb2.at[b]], rows2.at[b], gs[b])

    fire(0, 0)

    def accum_pair(k2, carry):
        for b in range(2):
            hc = k2 * 2 + b
            pltpu.make_async_copy(h_hbm.at[idxb2.at[b]], rows2.at[b],
                                  gs[b]).wait()

            @pl.when(hc + 1 < NHC)
            def _():
                fire(hc + 1, 1 - b)

            def group(g, c2):
                d16 = jnp.minimum(pk2[b, pl.ds(g * 16, 16)] >> 14, OWN)
                for jj in range(16):
                    d = d16[jj]
                    j = g * 16 + jj
                    for kk in range(HID // 16):
                        sl = pl.ds(kk * 16, 16)
                        plsc.addupdate(acc.at[d, sl], rows2[b, j, sl])
                return c2

            lax.fori_loop(0, HC // 16, group, 0)
        return carry

    lax.fori_loop(0, NHC // 2, accum_pair, 0)

    @pl.when(cid == 0)
    def _():
        @pl.when(sid < NSUB - 1)
        def _():
            pltpu.sync_copy(acc.at[pl.ds(0, OWN)],
                            agg0.at[pl.ds(row0, OWN)])

        @pl.when(sid == NSUB - 1)
        def _():
            pltpu.sync_copy(acc.at[pl.ds(0, N - (NSUB - 1) * OWN)],
                            agg0.at[pl.ds((NSUB - 1) * OWN,
                                          N - (NSUB - 1) * OWN)])

    @pl.when(cid == 1)
    def _():
        @pl.when(sid < NSUB - 1)
        def _():
            pltpu.sync_copy(acc.at[pl.ds(0, OWN)],
                            agg1.at[pl.ds(row0, OWN)])

        @pl.when(sid == NSUB - 1)
        def _():
            pltpu.sync_copy(acc.at[pl.ds(0, N - (NSUB - 1) * OWN)],
                            agg1.at[pl.ds((NSUB - 1) * OWN,
                                          N - (NSUB - 1) * OWN)])


@functools.cache
def _make_agg():
    return pl.kernel(
        _agg_body,
        out_type=[
            jax.ShapeDtypeStruct((N, HID), jnp.float32),
            jax.ShapeDtypeStruct((N, HID), jnp.float32),
        ],
        mesh=plsc.VectorSubcoreMesh(core_axis_name="c", subcore_axis_name="s"),
        scratch_types=[
            pltpu.VMEM((ECH,), jnp.int32),
            pltpu.VMEM((ECH,), jnp.int32),
            pltpu.VMEM((16, CH), jnp.int32),
            pltpu.VMEM((16, CH), jnp.int32),
            pltpu.VMEM((2, HC), jnp.int32),
            pltpu.VMEM((2, HC), jnp.int32),
            pltpu.VMEM((512,), jnp.int32),
            pltpu.VMEM((2, HC, HID), jnp.float32),
            pltpu.VMEM((OWN + 1, HID), jnp.float32),
            pltpu.VMEM_SHARED((LISTC,), jnp.int32),
            pltpu.SemaphoreType.DMA,
            pltpu.SemaphoreType.DMA,
            pltpu.SemaphoreType.DMA,
            pltpu.SemaphoreType.DMA,
            pltpu.SemaphoreType.DMA,
            pltpu.SemaphoreType.DMA,
            pltpu.SemaphoreType.DMA,
        ],
    )


def _dot(a, b):
    return jnp.dot(a, b, preferred_element_type=jnp.float32,
                   precision=lax.Precision.HIGHEST)


def _mlp_block(z, w1, b1, w2, b2, g, be):
    z = jnp.maximum(_dot(z, w1) + b1, 0.0)
    z = jnp.maximum(_dot(z, w2) + b2, 0.0)
    return z * g + be


def _layer_body(h_ref, a0_ref, a1_ref, w1_ref, b1_ref, w2_ref, b2_ref,
                g_ref, be_ref, o_ref):
    z = h_ref[...] + a0_ref[...] + a1_ref[...]
    o_ref[...] = _mlp_block(z, w1_ref[...], b1_ref[...], w2_ref[...],
                            b2_ref[...], g_ref[...], be_ref[...])


def _row_spec():
    return pl.BlockSpec((BLK, HID), lambda i: (i, 0))


def _full_spec(shape):
    return pl.BlockSpec(shape, lambda i: (0,) * len(shape))


_layer_call = pl.pallas_call(
    _layer_body,
    grid=(NBLK,),
    in_specs=[_row_spec(), _row_spec(), _row_spec(),
              _full_spec((HID, HID)), _full_spec((1, HID)),
              _full_spec((HID, HID)), _full_spec((1, HID)),
              _full_spec((1, HID)), _full_spec((1, HID))],
    out_specs=_row_spec(),
    out_shape=jax.ShapeDtypeStruct((N, HID), jnp.float32),
)


def _final_body(h_ref, a0_ref, a1_ref, w1_ref, b1_ref, w2_ref, b2_ref,
                g_ref, be_ref, batch_ref, l1w_ref, l1b_ref, l2w_ref, l2b_ref,
                emb_ref, logp_ref, out_ref, pool_acc, cnt_acc):
    i = pl.program_id(0)

    @pl.when(i == 0)
    def _():
        pool_acc[...] = jnp.zeros_like(pool_acc)
        cnt_acc[...] = jnp.zeros_like(cnt_acc)

    z = h_ref[...] + a0_ref[...] + a1_ref[...]
    h3 = _mlp_block(z, w1_ref[...], b1_ref[...], w2_ref[...], b2_ref[...],
                    g_ref[...], be_ref[...])
    emb_ref[...] = h3

    bt = batch_ref[...].reshape(1, BLK)
    seg = lax.broadcasted_iota(jnp.int32, (NG, BLK), 0)
    mask = (bt == seg).astype(jnp.float32)
    pool_acc[...] += lax.dot_general(
        mask, h3, (((1,), (0,)), ((), ())),
        preferred_element_type=jnp.float32, precision=lax.Precision.HIGHEST)
    cnt_acc[...] += jnp.broadcast_to(
        jnp.sum(mask, axis=1, keepdims=True), (NG, HID))

    @pl.when(i == NBLK - 1)
    def _():
        pooled = pool_acc[...] / jnp.maximum(cnt_acc[...], 1.0)
        z1 = _dot(pooled, l1w_ref[...]) + l1b_ref[...]
        z2 = _dot(z1, l2w_ref[...]) + l2b_ref[...]
        outv = jnp.clip(z2, -10.0, 10.0)
        out_ref[...] = outv
        lane = lax.broadcasted_iota(jnp.int32, (NG, HID), 1)
        masked = jnp.where(lane < NC, outv, -1e30)
        m = jnp.max(masked, axis=1, keepdims=True)
        lse = jnp.log(jnp.sum(jnp.exp(masked - m), axis=1, keepdims=True)) + m
        logp_ref[...] = outv - lse


_final_call = pl.pallas_call(
    _final_body,
    grid=(NBLK,),
    in_specs=[_row_spec(), _row_spec(), _row_spec(),
              _full_spec((HID, HID)), _full_spec((1, HID)),
              _full_spec((HID, HID)), _full_spec((1, HID)),
              _full_spec((1, HID)), _full_spec((1, HID)),
              pl.BlockSpec((1, 1, BLK), lambda i: (i, 0, 0)),
              _full_spec((HID, HID)), _full_spec((1, HID)),
              _full_spec((HID, HID)), _full_spec((1, HID))],
    out_specs=[_row_spec(), _full_spec((NG, HID)), _full_spec((NG, HID))],
    out_shape=[jax.ShapeDtypeStruct((N, HID), jnp.float32),
               jax.ShapeDtypeStruct((NG, HID), jnp.float32),
               jax.ShapeDtypeStruct((NG, HID), jnp.float32)],
    scratch_shapes=[pltpu.VMEM((NG, HID), jnp.float32),
                    pltpu.VMEM((NG, HID), jnp.float32)],
)

_BN_INV = 1.0 / (1.0 + BN_EPS) ** 0.5


def kernel(x, edge_index, batch, params):
    src = edge_index[0]
    dst = edge_index[1]
    npad = E_PAD - E
    pad_idx = jnp.arange(npad, dtype=jnp.int32)
    src_p = jnp.concatenate([src, pad_idx % N])
    dst_p = jnp.concatenate([dst, N + (pad_idx % PAD_ROWS)])
    batch3 = batch.reshape(NBLK, 1, BLK)

    def layer_args(p):
        return (p['W1'], p['b1'].reshape(1, HID), p['W2'],
                p['b2'].reshape(1, HID),
                (p['gamma'] * _BN_INV).reshape(1, HID),
                p['beta'].reshape(1, HID))

    h = x
    for i in range(NL - 1):
        a0, a1 = _make_agg()(h, src_p, dst_p)
        h = _layer_call(h, a0, a1, *layer_args(params['c%d' % i]))

    a0, a1 = _make_agg()(h, src_p, dst_p)
    l2w = jnp.zeros((HID, HID), jnp.float32).at[:, :NC].set(params['lin2_W'])
    l2b = jnp.zeros((1, HID), jnp.float32).at[0, :NC].set(params['lin2_b'])
    embeds, logp128, out128 = _final_call(
        h, a0, a1, *layer_args(params['c%d' % (NL - 1)]),
        batch3, params['lin1_W'], params['lin1_b'].reshape(1, HID), l2w, l2b)
    return (logp128[:, :NC], embeds, out128[:, :NC])


# speedup vs baseline: 1.5810x; 1.1349x over previous
import functools

import jax
import jax.numpy as jnp
from jax import lax
from jax.experimental import pallas as pl
from jax.experimental.pallas import tpu as pltpu
from jax.experimental.pallas import tpu_sc as plsc

N, E, NF, HID, NC, NL, NG = 10000, 320000, 128, 128, 10, 3, 64
BN_EPS = 1e-5

BLK = 1000
NBLK = N // BLK

NCORE, NSUB = 2, 16
NWORK = NCORE * NSUB
CH = 128
E_PAD = 327680
E_HALF = E_PAD // NCORE
PAD_ROWS = 16
OWN = 640
ECH = 2048
NECH = E_HALF // ECH
PLCAP = 896
NSLOT = 16 * PLCAP
CAPT = NSLOT + 16
NCHT = NSLOT // CH
HC = 64
NHC = NSLOT // HC
LISTC = NSUB * CAPT
SMASK = (1 << 14) - 1


def _agg_body(h_hbm, src_hbm, dst_hbm, agg0, agg1,
              esrc2, edst2, pos_b, val_b, pk2, idxb2, patt, rows2, acc, list_sh,
              gs0, gs1, es0, es1, psem, s0, s1, s2, s3):
    cid = lax.axis_index("c")
    sid = lax.axis_index("s")
    row0 = sid * OWN
    coff = cid * E_HALF
    tbase = sid * CAPT
    sems = [s0, s1, s2, s3]
    gs = [gs0, gs1]
    es = [es0, es1]

    zero16 = jnp.zeros((16,), jnp.float32)
    lanes = lax.broadcasted_iota(jnp.int32, (16,), 0)

    def zrow(r, carry):
        for kk in range(HID // 16):
            acc[r, pl.ds(kk * 16, 16)] = zero16
        return carry

    lax.fori_loop(0, OWN + 1, zrow, 0)

    def prow(t, carry):
        v = (((t * 16 + lanes) * 19) & 8191) | (OWN << 14)
        patt[pl.ds(t * 16, 16)] = v
        return carry

    lax.fori_loop(0, 32, prow, 0)
    pre = []
    for t in range(NSLOT // 512):
        pre.append(pltpu.async_copy(
            patt, list_sh.at[pl.ds(tbase + t * 512, 512)], psem))
    for cp in pre:
        cp.wait()

    def eload(it, b):
        off = coff + it * ECH
        pltpu.async_copy(src_hbm.at[pl.ds(off, ECH)], esrc2.at[b], es[b])
        pltpu.async_copy(dst_hbm.at[pl.ds(off, ECH)], edst2.at[b], es[b])

    eload(0, 0)

    def scan_pair(q, cnts):
        for b in range(2):
            it = q * 2 + b
            pltpu.make_async_copy(src_hbm.at[pl.ds(0, ECH)], esrc2.at[b],
                                  es[b]).wait()
            pltpu.make_async_copy(dst_hbm.at[pl.ds(0, ECH)], edst2.at[b],
                                  es[b]).wait()

            @pl.when(it + 1 < NECH)
            def _():
                eload(it + 1, 1 - b)

            for sc in range(4):
                for r in range(4):
                    @pl.when(it > 0)
                    def _():
                        pltpu.make_async_copy(
                            val_b.at[4 * sc + r],
                            list_sh.at[pos_b.at[4 * sc + r]],
                            sems[sc]).wait()

                def step(i4, cnts):
                    for u in range(4):
                        base_e = sc * 512 + (i4 * 4 + u) * 16
                        s16 = esrc2[b, pl.ds(base_e, 16)]
                        local = edst2[b, pl.ds(base_e, 16)] - row0
                        m = (local >= 0) & (local < OWN)
                        pos16 = jnp.where(m, tbase + lanes * PLCAP + cnts,
                                          tbase + NSLOT + lanes)
                        packed = s16 | (jnp.where(m, local, OWN) << 14)
                        row = 4 * sc + (i4 * 4 + u) // 8
                        col = ((i4 * 4 + u) % 8) * 16
                        pos_b[row, pl.ds(col, 16)] = pos16
                        val_b[row, pl.ds(col, 16)] = packed
                        cnts = cnts + jnp.where(m, 1, 0)
                    return cnts

                cnts = lax.fori_loop(0, 8, step, cnts)
                for r in range(4):
                    pltpu.async_copy(val_b.at[4 * sc + r],
                                     list_sh.at[pos_b.at[4 * sc + r]],
                                     sems[sc])
        return cnts

    lax.fori_loop(0, NECH // 2, scan_pair, jnp.zeros((16,), jnp.int32))
    for sc in range(4):
        for r in range(4):
            pltpu.make_async_copy(val_b.at[4 * sc + r],
                                  list_sh.at[pos_b.at[4 * sc + r]],
                                  sems[sc]).wait()

    def fire(hc, b):
        base = tbase + hc * HC
        pltpu.sync_copy(list_sh.at[pl.ds(base, HC)], pk2.at[b])
        for o in range(HC // 16):
            idxb2[b, pl.ds(o * 16, 16)] = pk2[b, pl.ds(o * 16, 16)] & SMASK
        pltpu.async_copy(h_hbm.at[idxb2.at[b]], rows2.at[b], gs[b])

    fire(0, 0)

    def accum_pair(k2, carry):
        for b in range(2):
            hc = k2 * 2 + b
            pltpu.make_async_copy(h_hbm.at[idxb2.at[b]], rows2.at[b],
                                  gs[b]).wait()

            @pl.when(hc + 1 < NHC)
            def _():
                fire(hc + 1, 1 - b)

            def group(g, c2):
                d16 = jnp.minimum(pk2[b, pl.ds(g * 16, 16)] >> 14, OWN)
                for jj in range(16):
                    d = d16[jj]
                    j = g * 16 + jj
                    for kk in range(HID // 16):
                        sl = pl.ds(kk * 16, 16)
                        plsc.addupdate(acc.at[d, sl], rows2[b, j, sl])
                return c2

            lax.fori_loop(0, HC // 16, group, 0)
        return carry

    lax.fori_loop(0, NHC // 2, accum_pair, 0)

    @pl.when(cid == 0)
    def _():
        @pl.when(sid < NSUB - 1)
        def _():
            pltpu.sync_copy(acc.at[pl.ds(0, OWN)],
                            agg0.at[pl.ds(row0, OWN)])

        @pl.when(sid == NSUB - 1)
        def _():
            pltpu.sync_copy(acc.at[pl.ds(0, N - (NSUB - 1) * OWN)],
                            agg0.at[pl.ds((NSUB - 1) * OWN,
                                          N - (NSUB - 1) * OWN)])

    @pl.when(cid == 1)
    def _():
        @pl.when(sid < NSUB - 1)
        def _():
            pltpu.sync_copy(acc.at[pl.ds(0, OWN)],
                            agg1.at[pl.ds(row0, OWN)])

        @pl.when(sid == NSUB - 1)
        def _():
            pltpu.sync_copy(acc.at[pl.ds(0, N - (NSUB - 1) * OWN)],
                            agg1.at[pl.ds((NSUB - 1) * OWN,
                                          N - (NSUB - 1) * OWN)])


@functools.cache
def _make_agg():
    return pl.kernel(
        _agg_body,
        out_type=[
            jax.ShapeDtypeStruct((N, HID), jnp.float32),
            jax.ShapeDtypeStruct((N, HID), jnp.float32),
        ],
        mesh=plsc.VectorSubcoreMesh(core_axis_name="c", subcore_axis_name="s"),
        scratch_types=[
            pltpu.VMEM((2, ECH), jnp.int32),
            pltpu.VMEM((2, ECH), jnp.int32),
            pltpu.VMEM((16, CH), jnp.int32),
            pltpu.VMEM((16, CH), jnp.int32),
            pltpu.VMEM((2, HC), jnp.int32),
            pltpu.VMEM((2, HC), jnp.int32),
            pltpu.VMEM((512,), jnp.int32),
            pltpu.VMEM((2, HC, HID), jnp.float32),
            pltpu.VMEM((OWN + 1, HID), jnp.float32),
            pltpu.VMEM_SHARED((LISTC,), jnp.int32),
            pltpu.SemaphoreType.DMA,
            pltpu.SemaphoreType.DMA,
            pltpu.SemaphoreType.DMA,
            pltpu.SemaphoreType.DMA,
            pltpu.SemaphoreType.DMA,
            pltpu.SemaphoreType.DMA,
            pltpu.SemaphoreType.DMA,
            pltpu.SemaphoreType.DMA,
            pltpu.SemaphoreType.DMA,
        ],
    )


def _dot(a, b):
    return jnp.dot(a, b, preferred_element_type=jnp.float32,
                   precision=lax.Precision.HIGHEST)


def _mlp_block(z, w1, b1, w2, b2, g, be):
    z = jnp.maximum(_dot(z, w1) + b1, 0.0)
    z = jnp.maximum(_dot(z, w2) + b2, 0.0)
    return z * g + be


def _layer_body(h_ref, a0_ref, a1_ref, w1_ref, b1_ref, w2_ref, b2_ref,
                g_ref, be_ref, o_ref):
    z = h_ref[...] + a0_ref[...] + a1_ref[...]
    o_ref[...] = _mlp_block(z, w1_ref[...], b1_ref[...], w2_ref[...],
                            b2_ref[...], g_ref[...], be_ref[...])


def _row_spec():
    return pl.BlockSpec((BLK, HID), lambda i: (i, 0))


def _full_spec(shape):
    return pl.BlockSpec(shape, lambda i: (0,) * len(shape))


_layer_call = pl.pallas_call(
    _layer_body,
    grid=(NBLK,),
    in_specs=[_row_spec(), _row_spec(), _row_spec(),
              _full_spec((HID, HID)), _full_spec((1, HID)),
              _full_spec((HID, HID)), _full_spec((1, HID)),
              _full_spec((1, HID)), _full_spec((1, HID))],
    out_specs=_row_spec(),
    out_shape=jax.ShapeDtypeStruct((N, HID), jnp.float32),
)


def _final_body(h_ref, a0_ref, a1_ref, w1_ref, b1_ref, w2_ref, b2_ref,
                g_ref, be_ref, batch_ref, l1w_ref, l1b_ref, l2w_ref, l2b_ref,
                emb_ref, logp_ref, out_ref, pool_acc, cnt_acc):
    i = pl.program_id(0)

    @pl.when(i == 0)
    def _():
        pool_acc[...] = jnp.zeros_like(pool_acc)
        cnt_acc[...] = jnp.zeros_like(cnt_acc)

    z = h_ref[...] + a0_ref[...] + a1_ref[...]
    h3 = _mlp_block(z, w1_ref[...], b1_ref[...], w2_ref[...], b2_ref[...],
                    g_ref[...], be_ref[...])
    emb_ref[...] = h3

    bt = batch_ref[...].reshape(1, BLK)
    seg = lax.broadcasted_iota(jnp.int32, (NG, BLK), 0)
    mask = (bt == seg).astype(jnp.float32)
    pool_acc[...] += lax.dot_general(
        mask, h3, (((1,), (0,)), ((), ())),
        preferred_element_type=jnp.float32, precision=lax.Precision.HIGHEST)
    cnt_acc[...] += jnp.broadcast_to(
        jnp.sum(mask, axis=1, keepdims=True), (NG, HID))

    @pl.when(i == NBLK - 1)
    def _():
        pooled = pool_acc[...] / jnp.maximum(cnt_acc[...], 1.0)
        z1 = _dot(pooled, l1w_ref[...]) + l1b_ref[...]
        z2 = _dot(z1, l2w_ref[...]) + l2b_ref[...]
        outv = jnp.clip(z2, -10.0, 10.0)
        out_ref[...] = outv
        lane = lax.broadcasted_iota(jnp.int32, (NG, HID), 1)
        masked = jnp.where(lane < NC, outv, -1e30)
        m = jnp.max(masked, axis=1, keepdims=True)
        lse = jnp.log(jnp.sum(jnp.exp(masked - m), axis=1, keepdims=True)) + m
        logp_ref[...] = outv - lse


_final_call = pl.pallas_call(
    _final_body,
    grid=(NBLK,),
    in_specs=[_row_spec(), _row_spec(), _row_spec(),
              _full_spec((HID, HID)), _full_spec((1, HID)),
              _full_spec((HID, HID)), _full_spec((1, HID)),
              _full_spec((1, HID)), _full_spec((1, HID)),
              pl.BlockSpec((1, 1, BLK), lambda i: (i, 0, 0)),
              _full_spec((HID, HID)), _full_spec((1, HID)),
              _full_spec((HID, HID)), _full_spec((1, HID))],
    out_specs=[_row_spec(), _full_spec((NG, HID)), _full_spec((NG, HID))],
    out_shape=[jax.ShapeDtypeStruct((N, HID), jnp.float32),
               jax.ShapeDtypeStruct((NG, HID), jnp.float32),
               jax.ShapeDtypeStruct((NG, HID), jnp.float32)],
    scratch_shapes=[pltpu.VMEM((NG, HID), jnp.float32),
                    pltpu.VMEM((NG, HID), jnp.float32)],
)

_BN_INV = 1.0 / (1.0 + BN_EPS) ** 0.5


def kernel(x, edge_index, batch, params):
    src = edge_index[0]
    dst = edge_index[1]
    npad = E_PAD - E
    pad_idx = jnp.arange(npad, dtype=jnp.int32)
    src_p = jnp.concatenate([src, pad_idx % N])
    dst_p = jnp.concatenate([dst, N + (pad_idx % PAD_ROWS)])
    batch3 = batch.reshape(NBLK, 1, BLK)

    def layer_args(p):
        return (p['W1'], p['b1'].reshape(1, HID), p['W2'],
                p['b2'].reshape(1, HID),
                (p['gamma'] * _BN_INV).reshape(1, HID),
                p['beta'].reshape(1, HID))

    h = x
    for i in range(NL - 1):
        a0, a1 = _make_agg()(h, src_p, dst_p)
        h = _layer_call(h, a0, a1, *layer_args(params['c%d' % i]))

    a0, a1 = _make_agg()(h, src_p, dst_p)
    l2w = jnp.zeros((HID, HID), jnp.float32).at[:, :NC].set(params['lin2_W'])
    l2b = jnp.zeros((1, HID), jnp.float32).at[0, :NC].set(params['lin2_b'])
    embeds, logp128, out128 = _final_call(
        h, a0, a1, *layer_args(params['c%d' % (NL - 1)]),
        batch3, params['lin1_W'], params['lin1_b'].reshape(1, HID), l2w, l2b)
    return (logp128[:, :NC], embeds, out128[:, :NC])


# speedup vs baseline: 1.6479x; 1.0423x over previous
import functools

import jax
import jax.numpy as jnp
from jax import lax
from jax.experimental import pallas as pl
from jax.experimental.pallas import tpu as pltpu
from jax.experimental.pallas import tpu_sc as plsc

N, E, NF, HID, NC, NL, NG = 10000, 320000, 128, 128, 10, 3, 64
BN_EPS = 1e-5

BLK = 1000
NBLK = N // BLK

NCORE, NSUB = 2, 16
NWORK = NCORE * NSUB
CH = 128
E_PAD = 327680
E_HALF = E_PAD // NCORE
PAD_ROWS = 16
OWN = 640
ECH = 2048
NECH = E_HALF // ECH
PLCAP = 896
NSLOT = 16 * PLCAP
CAPT = NSLOT + 16
NCHT = NSLOT // CH
HC = 64
NHC = NSLOT // HC
LISTC = NSUB * CAPT
SMASK = (1 << 14) - 1


def _agg_body(h_hbm, src_hbm, dst_hbm, agg0, agg1,
              esrc2, edst2, pos_b, val_b, pk2, idxb2, patt, rows2, acc, list_sh,
              gs0, gs1, es0, es1, psem, s0, s1, s2, s3):
    cid = lax.axis_index("c")
    sid = lax.axis_index("s")
    row0 = sid * OWN
    coff = cid * E_HALF
    tbase = sid * CAPT
    sems = [s0, s1, s2, s3]
    gs = [gs0, gs1]
    es = [es0, es1]

    zero16 = jnp.zeros((16,), jnp.float32)
    lanes = lax.broadcasted_iota(jnp.int32, (16,), 0)

    def zrow(r, carry):
        for kk in range(HID // 16):
            acc[r, pl.ds(kk * 16, 16)] = zero16
        return carry

    lax.fori_loop(0, OWN + 1, zrow, 0)

    def prow(t, carry):
        v = (((t * 16 + lanes) * 19) & 8191) | (OWN << 14)
        patt[pl.ds(t * 16, 16)] = v
        return carry

    lax.fori_loop(0, 32, prow, 0)
    pre = []
    for t in range(NSLOT // 512):
        pre.append(pltpu.async_copy(
            patt, list_sh.at[pl.ds(tbase + t * 512, 512)], psem))
    for cp in pre:
        cp.wait()

    def eload(it, b):
        off = coff + it * ECH
        pltpu.async_copy(src_hbm.at[pl.ds(off, ECH)], esrc2.at[b], es[b])
        pltpu.async_copy(dst_hbm.at[pl.ds(off, ECH)], edst2.at[b], es[b])

    eload(0, 0)

    def scan_pair(q, cnts):
        for b in range(2):
            it = q * 2 + b
            pltpu.make_async_copy(src_hbm.at[pl.ds(0, ECH)], esrc2.at[b],
                                  es[b]).wait()
            pltpu.make_async_copy(dst_hbm.at[pl.ds(0, ECH)], edst2.at[b],
                                  es[b]).wait()

            @pl.when(it + 1 < NECH)
            def _():
                eload(it + 1, 1 - b)

            for sc in range(4):
                for r in range(4):
                    @pl.when(it > 0)
                    def _():
                        pltpu.make_async_copy(
                            val_b.at[4 * sc + r],
                            list_sh.at[pos_b.at[4 * sc + r]],
                            sems[sc]).wait()

                def step(i4, cnts):
                    for u in range(4):
                        base_e = sc * 512 + (i4 * 4 + u) * 16
                        s16 = esrc2[b, pl.ds(base_e, 16)]
                        local = edst2[b, pl.ds(base_e, 16)] - row0
                        m = (local >= 0) & (local < OWN)
                        pos16 = jnp.where(m, tbase + lanes * PLCAP + cnts,
                                          tbase + NSLOT + lanes)
                        packed = s16 | (jnp.where(m, local, OWN) << 14)
                        row = 4 * sc + (i4 * 4 + u) // 8
                        col = ((i4 * 4 + u) % 8) * 16
                        pos_b[row, pl.ds(col, 16)] = pos16
                        val_b[row, pl.ds(col, 16)] = packed
                        cnts = cnts + jnp.where(m, 1, 0)
                    return cnts

                cnts = lax.fori_loop(0, 8, step, cnts)
                for r in range(4):
                    pltpu.async_copy(val_b.at[4 * sc + r],
                                     list_sh.at[pos_b.at[4 * sc + r]],
                                     sems[sc])
        return cnts

    lax.fori_loop(0, NECH // 2, scan_pair, jnp.zeros((16,), jnp.int32))
    for sc in range(4):
        for r in range(4):
            pltpu.make_async_copy(val_b.at[4 * sc + r],
                                  list_sh.at[pos_b.at[4 * sc + r]],
                                  sems[sc]).wait()

    def fire(hc, b):
        base = tbase + hc * HC
        pltpu.sync_copy(list_sh.at[pl.ds(base, HC)], pk2.at[b])
        for o in range(HC // 16):
            idxb2[b, pl.ds(o * 16, 16)] = pk2[b, pl.ds(o * 16, 16)] & SMASK
        pltpu.async_copy(h_hbm.at[idxb2.at[b]], rows2.at[b], gs[b])

    fire(0, 0)

    def accum_pair(k2, carry):
        for b in range(2):
            hc = k2 * 2 + b
            pltpu.make_async_copy(h_hbm.at[idxb2.at[b]], rows2.at[b],
                                  gs[b]).wait()

            @pl.when(hc + 1 < NHC)
            def _():
                fire(hc + 1, 1 - b)

            def group(g, c2):
                d16 = jnp.minimum(pk2[b, pl.ds(g * 16, 16)] >> 14, OWN)
                for jj in range(16):
                    d = d16[jj]
                    j = g * 16 + jj
                    for kk in range(HID // 16):
                        sl = pl.ds(kk * 16, 16)
                        plsc.addupdate(acc.at[d, sl], rows2[b, j, sl])
                return c2

            lax.fori_loop(0, HC // 16, group, 0)
        return carry

    lax.fori_loop(0, NHC // 2, accum_pair, 0)

    @pl.when(cid == 0)
    def _():
        @pl.when(sid < NSUB - 1)
        def _():
            pltpu.sync_copy(acc.at[pl.ds(0, OWN)],
                            agg0.at[pl.ds(row0, OWN)])

        @pl.when(sid == NSUB - 1)
        def _():
            pltpu.sync_copy(acc.at[pl.ds(0, N - (NSUB - 1) * OWN)],
                            agg0.at[pl.ds((NSUB - 1) * OWN,
                                          N - (NSUB - 1) * OWN)])

    @pl.when(cid == 1)
    def _():
        @pl.when(sid < NSUB - 1)
        def _():
            pltpu.sync_copy(acc.at[pl.ds(0, OWN)],
                            agg1.at[pl.ds(row0, OWN)])

        @pl.when(sid == NSUB - 1)
        def _():
            pltpu.sync_copy(acc.at[pl.ds(0, N - (NSUB - 1) * OWN)],
                            agg1.at[pl.ds((NSUB - 1) * OWN,
                                          N - (NSUB - 1) * OWN)])


@functools.cache
def _make_agg():
    return pl.kernel(
        _agg_body,
        out_type=[
            jax.ShapeDtypeStruct((N, HID), jnp.float32),
            jax.ShapeDtypeStruct((N, HID), jnp.float32),
        ],
        mesh=plsc.VectorSubcoreMesh(core_axis_name="c", subcore_axis_name="s"),
        scratch_types=[
            pltpu.VMEM((2, ECH), jnp.int32),
            pltpu.VMEM((2, ECH), jnp.int32),
            pltpu.VMEM((16, CH), jnp.int32),
            pltpu.VMEM((16, CH), jnp.int32),
            pltpu.VMEM((2, HC), jnp.int32),
            pltpu.VMEM((2, HC), jnp.int32),
            pltpu.VMEM((512,), jnp.int32),
            pltpu.VMEM((2, HC, HID), jnp.float32),
            pltpu.VMEM((OWN + 1, HID), jnp.float32),
            pltpu.VMEM_SHARED((LISTC,), jnp.int32),
            pltpu.SemaphoreType.DMA,
            pltpu.SemaphoreType.DMA,
            pltpu.SemaphoreType.DMA,
            pltpu.SemaphoreType.DMA,
            pltpu.SemaphoreType.DMA,
            pltpu.SemaphoreType.DMA,
            pltpu.SemaphoreType.DMA,
            pltpu.SemaphoreType.DMA,
            pltpu.SemaphoreType.DMA,
        ],
    )


def _dot(a, b):
    return jnp.dot(a, b, preferred_element_type=jnp.float32)


def _mlp_block(z, w1, b1, w2, b2, g, be):
    z = jnp.maximum(_dot(z, w1) + b1, 0.0)
    z = jnp.maximum(_dot(z, w2) + b2, 0.0)
    return z * g + be


def _layer_body(h_ref, a0_ref, a1_ref, w1_ref, b1_ref, w2_ref, b2_ref,
                g_ref, be_ref, o_ref):
    z = h_ref[...] + a0_ref[...] + a1_ref[...]
    o_ref[...] = _mlp_block(z, w1_ref[...], b1_ref[...], w2_ref[...],
                            b2_ref[...], g_ref[...], be_ref[...])


def _row_spec():
    return pl.BlockSpec((BLK, HID), lambda i: (i, 0))


def _full_spec(shape):
    return pl.BlockSpec(shape, lambda i: (0,) * len(shape))


_layer_call = pl.pallas_call(
    _layer_body,
    grid=(NBLK,),
    in_specs=[_row_spec(), _row_spec(), _row_spec(),
              _full_spec((HID, HID)), _full_spec((1, HID)),
              _full_spec((HID, HID)), _full_spec((1, HID)),
              _full_spec((1, HID)), _full_spec((1, HID))],
    out_specs=_row_spec(),
    out_shape=jax.ShapeDtypeStruct((N, HID), jnp.float32),
)


def _final_body(h_ref, a0_ref, a1_ref, w1_ref, b1_ref, w2_ref, b2_ref,
                g_ref, be_ref, batch_ref, l1w_ref, l1b_ref, l2w_ref, l2b_ref,
                emb_ref, logp_ref, out_ref, pool_acc, cnt_acc):
    i = pl.program_id(0)

    @pl.when(i == 0)
    def _():
        pool_acc[...] = jnp.zeros_like(pool_acc)
        cnt_acc[...] = jnp.zeros_like(cnt_acc)

    z = h_ref[...] + a0_ref[...] + a1_ref[...]
    h3 = _mlp_block(z, w1_ref[...], b1_ref[...], w2_ref[...], b2_ref[...],
                    g_ref[...], be_ref[...])
    emb_ref[...] = h3

    bt = batch_ref[...].reshape(1, BLK)
    seg = lax.broadcasted_iota(jnp.int32, (NG, BLK), 0)
    mask = (bt == seg).astype(jnp.float32)
    pool_acc[...] += lax.dot_general(
        mask, h3, (((1,), (0,)), ((), ())),
        preferred_element_type=jnp.float32)
    cnt_acc[...] += jnp.broadcast_to(
        jnp.sum(mask, axis=1, keepdims=True), (NG, HID))

    @pl.when(i == NBLK - 1)
    def _():
        pooled = pool_acc[...] / jnp.maximum(cnt_acc[...], 1.0)
        z1 = _dot(pooled, l1w_ref[...]) + l1b_ref[...]
        z2 = _dot(z1, l2w_ref[...]) + l2b_ref[...]
        outv = jnp.clip(z2, -10.0, 10.0)
        out_ref[...] = outv
        lane = lax.broadcasted_iota(jnp.int32, (NG, HID), 1)
        masked = jnp.where(lane < NC, outv, -1e30)
        m = jnp.max(masked, axis=1, keepdims=True)
        lse = jnp.log(jnp.sum(jnp.exp(masked - m), axis=1, keepdims=True)) + m
        logp_ref[...] = outv - lse


_final_call = pl.pallas_call(
    _final_body,
    grid=(NBLK,),
    in_specs=[_row_spec(), _row_spec(), _row_spec(),
              _full_spec((HID, HID)), _full_spec((1, HID)),
              _full_spec((HID, HID)), _full_spec((1, HID)),
              _full_spec((1, HID)), _full_spec((1, HID)),
              pl.BlockSpec((1, 1, BLK), lambda i: (i, 0, 0)),
              _full_spec((HID, HID)), _full_spec((1, HID)),
              _full_spec((HID, HID)), _full_spec((1, HID))],
    out_specs=[_row_spec(), _full_spec((NG, HID)), _full_spec((NG, HID))],
    out_shape=[jax.ShapeDtypeStruct((N, HID), jnp.float32),
               jax.ShapeDtypeStruct((NG, HID), jnp.float32),
               jax.ShapeDtypeStruct((NG, HID), jnp.float32)],
    scratch_shapes=[pltpu.VMEM((NG, HID), jnp.float32),
                    pltpu.VMEM((NG, HID), jnp.float32)],
)

_BN_INV = 1.0 / (1.0 + BN_EPS) ** 0.5


def kernel(x, edge_index, batch, params):
    src = edge_index[0]
    dst = edge_index[1]
    npad = E_PAD - E
    pad_idx = jnp.arange(npad, dtype=jnp.int32)
    src_p = jnp.concatenate([src, pad_idx % N])
    dst_p = jnp.concatenate([dst, N + (pad_idx % PAD_ROWS)])
    batch3 = batch.reshape(NBLK, 1, BLK)

    def layer_args(p):
        return (p['W1'], p['b1'].reshape(1, HID), p['W2'],
                p['b2'].reshape(1, HID),
                (p['gamma'] * _BN_INV).reshape(1, HID),
                p['beta'].reshape(1, HID))

    h = x
    for i in range(NL - 1):
        a0, a1 = _make_agg()(h, src_p, dst_p)
        h = _layer_call(h, a0, a1, *layer_args(params['c%d' % i]))

    a0, a1 = _make_agg()(h, src_p, dst_p)
    l2w = jnp.zeros((HID, HID), jnp.float32).at[:, :NC].set(params['lin2_W'])
    l2b = jnp.zeros((1, HID), jnp.float32).at[0, :NC].set(params['lin2_b'])
    embeds, logp128, out128 = _final_call(
        h, a0, a1, *layer_args(params['c%d' % (NL - 1)]),
        batch3, params['lin1_W'], params['lin1_b'].reshape(1, HID), l2w, l2b)
    return (logp128[:, :NC], embeds, out128[:, :NC])
